# Initial kernel scaffold; baseline (speedup 1.0000x reference)
import jax
import jax.numpy as jnp
from jax.experimental import pallas as pl


def kernel(x, edge_index, Wl1, bl1, Wr1, Wl2, bl2, Wr2):
    raise NotImplementedError("write your pallas kernel here")


# speedup vs baseline: 5.2433x; 5.2433x over previous
import functools

import jax
import jax.numpy as jnp
from jax import lax
from jax.experimental import pallas as pl
from jax.experimental.pallas import tpu as pltpu
from jax.experimental.pallas import tpu_sc as plsc

N = 10000
D = 128
E = 320000

NC = 2
NS = 16
NW = NC * NS

CH = 128
CPW = 79
EPW = CPW * CH
E_PAD = NW * EPW
NACC = 10016
RPT = NACC // NS


def _make_agg(do_deg):
    mesh = plsc.VectorSubcoreMesh(core_axis_name="c", subcore_axis_name="s")
    out_type = [jax.ShapeDtypeStruct((NC, NACC, D), jnp.float32)]
    scratch = [
        pltpu.VMEM((CPW, CH), jnp.int32),
        pltpu.VMEM((CPW, CH), jnp.int32),
        pltpu.VMEM((CH, D), jnp.float32),
        pltpu.VMEM_SHARED((NACC, D), jnp.float32),
        pltpu.SemaphoreType.DMA,
    ]
    if do_deg:
        out_type.append(jax.ShapeDtypeStruct((NC, NACC, 16), jnp.float32))
        scratch += [
            pltpu.VMEM((CH, 16), jnp.float32),
            pltpu.VMEM((CH, 16), jnp.float32),
            pltpu.VMEM_SHARED((NACC, 16), jnp.float32),
        ]

    @functools.partial(
        pl.kernel, mesh=mesh, out_type=out_type, scratch_types=scratch,
        compiler_params=pltpu.CompilerParams(use_tc_tiling_on_sc=False))
    def agg(*refs):
        if do_deg:
            (x_hbm, src_hbm, dst_hbm,
             agg_out, deg_out, src_v, dst_v, rows_v, acc_sh, sem,
             ones_v, zd_v, deg_sh) = refs
        else:
            (x_hbm, src_hbm, dst_hbm,
             agg_out, src_v, dst_v, rows_v, acc_sh, sem) = refs
        c = lax.axis_index("c")
        s = lax.axis_index("s")
        wid = c * NS + s

        z16 = jnp.zeros((16,), jnp.float32)
        o16 = jnp.ones((16,), jnp.float32)

        def fill(i, carry):
            for j in range(D // 16):
                rows_v[i, pl.ds(j * 16, 16)] = z16
            if do_deg:
                ones_v[i, :] = o16
                zd_v[i, :] = z16
            return carry

        lax.fori_loop(0, CH, fill, 0)

        for k in range(4):
            pltpu.sync_copy(rows_v, acc_sh.at[pl.ds(s * RPT + k * CH, CH)])
        pltpu.sync_copy(rows_v.at[pl.ds(0, RPT - 4 * CH)],
                        acc_sh.at[pl.ds(s * RPT + 4 * CH, RPT - 4 * CH)])
        if do_deg:
            for k in range(4):
                pltpu.sync_copy(zd_v, deg_sh.at[pl.ds(s * RPT + k * CH, CH)])
            pltpu.sync_copy(zd_v.at[pl.ds(0, RPT - 4 * CH)],
                            deg_sh.at[pl.ds(s * RPT + 4 * CH, RPT - 4 * CH)])
        pltpu.sync_copy(src_hbm.at[wid], src_v)
        pltpu.sync_copy(dst_hbm.at[wid], dst_v)
        plsc.subcore_barrier()

        def chunk(j, carry):
            pltpu.async_copy(x_hbm.at[src_v.at[j]], rows_v, sem).wait()
            pltpu.sync_copy(rows_v, acc_sh.at[dst_v.at[j]], add=True)
            if do_deg:
                pltpu.sync_copy(ones_v, deg_sh.at[dst_v.at[j]], add=True)
            return carry

        lax.fori_loop(0, CPW, chunk, 0)
        plsc.subcore_barrier()

        ob = s * RPT
        pltpu.sync_copy(acc_sh.at[pl.ds(ob, RPT)], agg_out.at[c, pl.ds(ob, RPT)])
        if do_deg:
            pltpu.sync_copy(deg_sh.at[pl.ds(ob, RPT)], deg_out.at[c, pl.ds(ob, RPT)])

    return agg


_agg_with_deg = _make_agg(True)
_agg_no_deg = _make_agg(False)

_BN = 1000


def _dense_body(relu, agg_ref, deg_ref, x_ref, wl_ref, bl_ref, wr_ref, o_ref):
    agg = agg_ref[0] + agg_ref[1]
    deg = deg_ref[0, :, 0:1] + deg_ref[1, :, 0:1]
    mean = agg / jnp.maximum(deg, 1.0)
    y = (jnp.dot(mean, wl_ref[...], preferred_element_type=jnp.float32)
         + jnp.dot(x_ref[...], wr_ref[...], preferred_element_type=jnp.float32)
         + bl_ref[...])
    o_ref[...] = jnp.maximum(y, 0.0) if relu else y


def _dense(aggp, degp, x, WlT, bl, WrT, relu):
    return pl.pallas_call(
        functools.partial(_dense_body, relu),
        grid=(N // _BN,),
        in_specs=[
            pl.BlockSpec((NC, _BN, D), lambda i: (0, i, 0)),
            pl.BlockSpec((NC, _BN, 16), lambda i: (0, i, 0)),
            pl.BlockSpec((_BN, D), lambda i: (i, 0)),
            pl.BlockSpec((D, D), lambda i: (0, 0)),
            pl.BlockSpec((1, D), lambda i: (0, 0)),
            pl.BlockSpec((D, D), lambda i: (0, 0)),
        ],
        out_specs=pl.BlockSpec((_BN, D), lambda i: (i, 0)),
        out_shape=jax.ShapeDtypeStruct((N, D), jnp.float32),
    )(aggp, degp, x, WlT, bl, WrT)


def kernel(x, edge_index, Wl1, bl1, Wr1, Wl2, bl2, Wr2):
    src = edge_index[0]
    dst = edge_index[1]
    pad = E_PAD - E
    src_p = jnp.concatenate([src, jnp.zeros((pad,), jnp.int32)]).reshape(NW, CPW, CH)
    dst_p = jnp.concatenate([dst, jnp.full((pad,), N, jnp.int32)]).reshape(NW, CPW, CH)
    agg1, deg = _agg_with_deg(x, src_p, dst_p)
    h1 = _dense(agg1, deg, x, Wl1.T, bl1.reshape(1, D), Wr1.T, relu=True)
    (agg2,) = _agg_no_deg(h1, src_p, dst_p)
    out = _dense(agg2, deg, h1, Wl2.T, bl2.reshape(1, D), Wr2.T, relu=False)
    return out


# speedup vs baseline: 6.1557x; 1.1740x over previous
import functools

import jax
import jax.numpy as jnp
from jax import lax
from jax.experimental import pallas as pl
from jax.experimental.pallas import tpu as pltpu
from jax.experimental.pallas import tpu_sc as plsc

N = 10000
D = 128
E = 320000

NC = 2
NS = 16
NW = NC * NS

CH = 64
CPW = 158
EPW = CPW * CH
E_PAD = NW * EPW
NACC = 10016
RPT = NACC // NS


def _make_agg(do_deg):
    mesh = plsc.VectorSubcoreMesh(core_axis_name="c", subcore_axis_name="s")
    out_type = [jax.ShapeDtypeStruct((NC, NACC, D), jnp.float32)]
    scratch = [
        pltpu.VMEM((CPW, CH), jnp.int32),
        pltpu.VMEM((CPW, CH), jnp.int32),
        pltpu.VMEM((CH, D), jnp.float32),
        pltpu.VMEM((CH, D), jnp.float32),
        pltpu.VMEM_SHARED((NACC, D), jnp.float32),
        pltpu.SemaphoreType.DMA,
        pltpu.SemaphoreType.DMA,
    ]
    if do_deg:
        out_type.append(jax.ShapeDtypeStruct((NC, NACC, 16), jnp.float32))
        scratch += [
            pltpu.VMEM((CH, 16), jnp.float32),
            pltpu.VMEM((CH, 16), jnp.float32),
            pltpu.VMEM_SHARED((NACC, 16), jnp.float32),
        ]

    @functools.partial(
        pl.kernel, mesh=mesh, out_type=out_type, scratch_types=scratch,
        compiler_params=pltpu.CompilerParams(use_tc_tiling_on_sc=False))
    def agg(*refs):
        if do_deg:
            (x_hbm, src_hbm, dst_hbm,
             agg_out, deg_out, src_v, dst_v, rows0, rows1, acc_sh, sem0, sem1,
             ones_v, zd_v, deg_sh) = refs
        else:
            (x_hbm, src_hbm, dst_hbm,
             agg_out, src_v, dst_v, rows0, rows1, acc_sh, sem0, sem1) = refs
        c = lax.axis_index("c")
        s = lax.axis_index("s")
        wid = c * NS + s

        z16 = jnp.zeros((16,), jnp.float32)
        o16 = jnp.ones((16,), jnp.float32)

        def fill(i, carry):
            for j in range(D // 16):
                rows0[i, pl.ds(j * 16, 16)] = z16
            if do_deg:
                ones_v[i, :] = o16
                zd_v[i, :] = z16
            return carry

        lax.fori_loop(0, CH, fill, 0)

        nfull = RPT // CH
        rem = RPT - nfull * CH
        for k in range(nfull):
            pltpu.sync_copy(rows0, acc_sh.at[pl.ds(s * RPT + k * CH, CH)])
        pltpu.sync_copy(rows0.at[pl.ds(0, rem)],
                        acc_sh.at[pl.ds(s * RPT + nfull * CH, rem)])
        if do_deg:
            for k in range(nfull):
                pltpu.sync_copy(zd_v, deg_sh.at[pl.ds(s * RPT + k * CH, CH)])
            pltpu.sync_copy(zd_v.at[pl.ds(0, rem)],
                            deg_sh.at[pl.ds(s * RPT + nfull * CH, rem)])
        pltpu.sync_copy(src_hbm.at[wid], src_v)
        pltpu.sync_copy(dst_hbm.at[wid], dst_v)
        plsc.subcore_barrier()

        def scat(buf, j):
            pltpu.sync_copy(buf, acc_sh.at[dst_v.at[j]], add=True)
            if do_deg:
                pltpu.sync_copy(ones_v, deg_sh.at[dst_v.at[j]], add=True)

        pltpu.async_copy(x_hbm.at[src_v.at[0]], rows0, sem0)

        def body(g, carry):
            j0 = 2 * g
            pltpu.async_copy(x_hbm.at[src_v.at[j0 + 1]], rows1, sem1)
            pltpu.make_async_copy(x_hbm.at[src_v.at[j0]], rows0, sem0).wait()
            scat(rows0, j0)
            pltpu.async_copy(x_hbm.at[src_v.at[j0 + 2]], rows0, sem0)
            pltpu.make_async_copy(x_hbm.at[src_v.at[j0 + 1]], rows1, sem1).wait()
            scat(rows1, j0 + 1)
            return carry

        lax.fori_loop(0, (CPW - 2) // 2, body, 0)
        pltpu.async_copy(x_hbm.at[src_v.at[CPW - 1]], rows1, sem1)
        pltpu.make_async_copy(x_hbm.at[src_v.at[CPW - 2]], rows0, sem0).wait()
        scat(rows0, CPW - 2)
        pltpu.make_async_copy(x_hbm.at[src_v.at[CPW - 1]], rows1, sem1).wait()
        scat(rows1, CPW - 1)
        plsc.subcore_barrier()

        ob = s * RPT
        pltpu.sync_copy(acc_sh.at[pl.ds(ob, RPT)], agg_out.at[c, pl.ds(ob, RPT)])
        if do_deg:
            pltpu.sync_copy(deg_sh.at[pl.ds(ob, RPT)], deg_out.at[c, pl.ds(ob, RPT)])

    return agg


_agg_with_deg = _make_agg(True)
_agg_no_deg = _make_agg(False)

_BN = 1000


def _dense_body(relu, agg_ref, deg_ref, x_ref, wl_ref, bl_ref, wr_ref, o_ref):
    agg = agg_ref[0] + agg_ref[1]
    deg = deg_ref[0, :, 0:1] + deg_ref[1, :, 0:1]
    mean = agg / jnp.maximum(deg, 1.0)
    y = (jnp.dot(mean, wl_ref[...], preferred_element_type=jnp.float32)
         + jnp.dot(x_ref[...], wr_ref[...], preferred_element_type=jnp.float32)
         + bl_ref[...])
    o_ref[...] = jnp.maximum(y, 0.0) if relu else y


def _dense(aggp, degp, x, WlT, bl, WrT, relu):
    return pl.pallas_call(
        functools.partial(_dense_body, relu),
        grid=(N // _BN,),
        in_specs=[
            pl.BlockSpec((NC, _BN, D), lambda i: (0, i, 0)),
            pl.BlockSpec((NC, _BN, 16), lambda i: (0, i, 0)),
            pl.BlockSpec((_BN, D), lambda i: (i, 0)),
            pl.BlockSpec((D, D), lambda i: (0, 0)),
            pl.BlockSpec((1, D), lambda i: (0, 0)),
            pl.BlockSpec((D, D), lambda i: (0, 0)),
        ],
        out_specs=pl.BlockSpec((_BN, D), lambda i: (i, 0)),
        out_shape=jax.ShapeDtypeStruct((N, D), jnp.float32),
    )(aggp, degp, x, WlT, bl, WrT)


def kernel(x, edge_index, Wl1, bl1, Wr1, Wl2, bl2, Wr2):
    src = edge_index[0]
    dst = edge_index[1]
    pad = E_PAD - E
    src_p = jnp.concatenate([src, jnp.zeros((pad,), jnp.int32)]).reshape(NW, CPW, CH)
    dst_p = jnp.concatenate([dst, jnp.full((pad,), N, jnp.int32)]).reshape(NW, CPW, CH)
    agg1, deg = _agg_with_deg(x, src_p, dst_p)
    h1 = _dense(agg1, deg, x, Wl1.T, bl1.reshape(1, D), Wr1.T, relu=True)
    (agg2,) = _agg_no_deg(h1, src_p, dst_p)
    out = _dense(agg2, deg, h1, Wl2.T, bl2.reshape(1, D), Wr2.T, relu=False)
    return out


# speedup vs baseline: 6.5132x; 1.0581x over previous
import functools

import jax
import jax.numpy as jnp
from jax import lax
from jax.experimental import pallas as pl
from jax.experimental.pallas import tpu as pltpu
from jax.experimental.pallas import tpu_sc as plsc

N = 10000
D = 128
E = 320000

NC = 2
NS = 16
NW = NC * NS

CH = 64
CPW = 158
EPW = CPW * CH
E_PAD = NW * EPW
NACC = 10016
RPT = NACC // NS


def _make_agg(do_deg):
    mesh = plsc.VectorSubcoreMesh(core_axis_name="c", subcore_axis_name="s")
    out_type = [jax.ShapeDtypeStruct((NC, NACC, D), jnp.float32)]
    scratch = [
        pltpu.VMEM((CPW, CH), jnp.int32),
        pltpu.VMEM((CPW, CH), jnp.int32),
        pltpu.VMEM((CH, D), jnp.float32),
        pltpu.VMEM((CH, D), jnp.float32),
        pltpu.VMEM_SHARED((NACC, D), jnp.float32),
        pltpu.SemaphoreType.DMA,
        pltpu.SemaphoreType.DMA,
    ]
    if do_deg:
        out_type.append(jax.ShapeDtypeStruct((NC, NACC, 16), jnp.float32))
        scratch += [
            pltpu.VMEM((CH, 16), jnp.float32),
            pltpu.VMEM((CH, 16), jnp.float32),
            pltpu.VMEM_SHARED((NACC, 16), jnp.float32),
        ]

    @functools.partial(
        pl.kernel, mesh=mesh, out_type=out_type, scratch_types=scratch,
        compiler_params=pltpu.CompilerParams(use_tc_tiling_on_sc=False))
    def agg(*refs):
        if do_deg:
            (x_hbm, src_hbm, dst_hbm,
             agg_out, deg_out, src_v, dst_v, rows0, rows1, acc_sh, sem0, sem1,
             ones_v, zd_v, deg_sh) = refs
        else:
            (x_hbm, src_hbm, dst_hbm,
             agg_out, src_v, dst_v, rows0, rows1, acc_sh, sem0, sem1) = refs
        c = lax.axis_index("c")
        s = lax.axis_index("s")
        wid = c * NS + s

        z16 = jnp.zeros((16,), jnp.float32)
        o16 = jnp.ones((16,), jnp.float32)

        def fill(i, carry):
            for j in range(D // 16):
                rows0[i, pl.ds(j * 16, 16)] = z16
            if do_deg:
                ones_v[i, :] = o16
                zd_v[i, :] = z16
            return carry

        lax.fori_loop(0, CH, fill, 0)

        nfull = RPT // CH
        rem = RPT - nfull * CH
        for k in range(nfull):
            pltpu.sync_copy(rows0, acc_sh.at[pl.ds(s * RPT + k * CH, CH)])
        pltpu.sync_copy(rows0.at[pl.ds(0, rem)],
                        acc_sh.at[pl.ds(s * RPT + nfull * CH, rem)])
        if do_deg:
            for k in range(nfull):
                pltpu.sync_copy(zd_v, deg_sh.at[pl.ds(s * RPT + k * CH, CH)])
            pltpu.sync_copy(zd_v.at[pl.ds(0, rem)],
                            deg_sh.at[pl.ds(s * RPT + nfull * CH, rem)])
        pltpu.sync_copy(src_hbm.at[wid], src_v)
        pltpu.sync_copy(dst_hbm.at[wid], dst_v)
        plsc.subcore_barrier()

        def scat(buf, j):
            pltpu.sync_copy(buf, acc_sh.at[dst_v.at[j]], add=True)
            if do_deg:
                pltpu.sync_copy(ones_v, deg_sh.at[dst_v.at[j]], add=True)

        pltpu.async_copy(x_hbm.at[src_v.at[0]], rows0, sem0)

        def body(g, carry):
            j0 = 2 * g
            pltpu.async_copy(x_hbm.at[src_v.at[j0 + 1]], rows1, sem1)
            pltpu.make_async_copy(x_hbm.at[src_v.at[j0]], rows0, sem0).wait()
            scat(rows0, j0)
            pltpu.async_copy(x_hbm.at[src_v.at[j0 + 2]], rows0, sem0)
            pltpu.make_async_copy(x_hbm.at[src_v.at[j0 + 1]], rows1, sem1).wait()
            scat(rows1, j0 + 1)
            return carry

        lax.fori_loop(0, (CPW - 2) // 2, body, 0)
        pltpu.async_copy(x_hbm.at[src_v.at[CPW - 1]], rows1, sem1)
        pltpu.make_async_copy(x_hbm.at[src_v.at[CPW - 2]], rows0, sem0).wait()
        scat(rows0, CPW - 2)
        pltpu.make_async_copy(x_hbm.at[src_v.at[CPW - 1]], rows1, sem1).wait()
        scat(rows1, CPW - 1)
        plsc.subcore_barrier()

        ob = s * RPT
        pltpu.sync_copy(acc_sh.at[pl.ds(ob, RPT)], agg_out.at[c, pl.ds(ob, RPT)])
        if do_deg:
            pltpu.sync_copy(deg_sh.at[pl.ds(ob, RPT)], deg_out.at[c, pl.ds(ob, RPT)])

    return agg


_agg_with_deg = _make_agg(True)
_agg_no_deg = _make_agg(False)

_BN = 1000


def _dense_body(relu, agg_ref, deg_ref, x_ref, wl_ref, bl_ref, wr_ref, o_ref):
    agg = agg_ref[0] + agg_ref[1]
    deg = deg_ref[0, :, 0:1] + deg_ref[1, :, 0:1]
    mean = agg / jnp.maximum(deg, 1.0)
    y = (jnp.dot(mean, wl_ref[...], preferred_element_type=jnp.float32)
         + jnp.dot(x_ref[...], wr_ref[...], preferred_element_type=jnp.float32)
         + bl_ref[...])
    o_ref[...] = jnp.maximum(y, 0.0) if relu else y


def _dense(aggp, degp, x, WlT, bl, WrT, relu):
    return pl.pallas_call(
        functools.partial(_dense_body, relu),
        grid=(N // _BN,),
        in_specs=[
            pl.BlockSpec((NC, _BN, D), lambda i: (0, i, 0)),
            pl.BlockSpec((NC, _BN, 16), lambda i: (0, i, 0)),
            pl.BlockSpec((_BN, D), lambda i: (i, 0)),
            pl.BlockSpec((D, D), lambda i: (0, 0)),
            pl.BlockSpec((1, D), lambda i: (0, 0)),
            pl.BlockSpec((D, D), lambda i: (0, 0)),
        ],
        out_specs=pl.BlockSpec((_BN, D), lambda i: (i, 0)),
        out_shape=jax.ShapeDtypeStruct((N, D), jnp.float32),
    )(aggp, degp, x, WlT, bl, WrT)


def kernel(x, edge_index, Wl1, bl1, Wr1, Wl2, bl2, Wr2):
    src = edge_index[0]
    dst = edge_index[1]
    epw = E // NW
    pad_w = EPW - epw
    src_p = jnp.concatenate(
        [src.reshape(NW, epw), jnp.zeros((NW, pad_w), jnp.int32)], axis=1
    ).reshape(NW, CPW, CH)
    dummy = N + (jnp.arange(pad_w, dtype=jnp.int32) % (NACC - N))
    dst_p = jnp.concatenate(
        [dst.reshape(NW, epw), jnp.broadcast_to(dummy, (NW, pad_w))], axis=1
    ).reshape(NW, CPW, CH)
    agg1, deg = _agg_with_deg(x, src_p, dst_p)
    h1 = _dense(agg1, deg, x, Wl1.T, bl1.reshape(1, D), Wr1.T, relu=True)
    (agg2,) = _agg_no_deg(h1, src_p, dst_p)
    out = _dense(agg2, deg, h1, Wl2.T, bl2.reshape(1, D), Wr2.T, relu=False)
    return out
